# Initial kernel scaffold; baseline (speedup 1.0000x reference)
import jax
import jax.numpy as jnp
from jax.experimental import pallas as pl


def kernel(hidden_states, gate_w, gate_b, w_gate_up, w_down, sh_gate_up, sh_down):
    raise NotImplementedError("write your pallas kernel here")


# speedup vs baseline: 2.2242x; 2.2242x over previous
import functools

import jax
import jax.numpy as jnp
from jax import lax
from jax.experimental import pallas as pl
from jax.experimental.pallas import tpu as pltpu
from jax.experimental.pallas import tpu_sc as plsc

D = 1024
DFF = 512
E = 64
NG = 8
GSZ = E // NG
TKG = 4
TOPK = 8
NSH = 2
RS = 2.5
T = 2048

TILE = 128
P_PAD = 16384 + E * TILE
NTILES = P_PAD // TILE
NW = 32
TPW = T // NW

_BIG = 1 << 30
_NEG = -1e30


def _gate_body(x_ref, gw_ref, gb_ref, idx_ref, w_ref):
    x = x_ref[...]
    z = lax.dot_general(x, gw_ref[...], (((1,), (1,)), ((), ())),
                        preferred_element_type=jnp.float32,
                        precision=lax.Precision.DEFAULT)
    scores = jax.nn.sigmoid(z)
    s = scores + gb_ref[...]

    col = lax.broadcasted_iota(jnp.int32, (T, E), 1)
    gid = col // GSZ

    gsc = jnp.zeros((T, NG), jnp.float32)
    gcol = lax.broadcasted_iota(jnp.int32, (T, NG), 1)
    for g in range(NG):
        m = jnp.where(gid == g, s, _NEG)
        m1 = jnp.max(m, axis=1, keepdims=True)
        i1 = jnp.min(jnp.where(m == m1, col, _BIG), axis=1, keepdims=True)
        m2 = jnp.max(jnp.where(col == i1, _NEG, m), axis=1, keepdims=True)
        gsc = gsc + jnp.where(gcol == g, m1 + m2, 0.0)

    sel_g = jnp.zeros((T, NG), jnp.bool_)
    gs = gsc
    for _ in range(TKG):
        mg = jnp.max(gs, axis=1, keepdims=True)
        ig = jnp.min(jnp.where(gs == mg, gcol, _BIG), axis=1, keepdims=True)
        sel_g = sel_g | (gcol == ig)
        gs = jnp.where(gcol == ig, _NEG, gs)

    allowed = jnp.zeros((T, E), jnp.bool_)
    for g in range(NG):
        allowed = allowed | ((gid == g) & sel_g[:, g:g + 1])

    sm = jnp.where(allowed, s, _NEG)
    kcol = lax.broadcasted_iota(jnp.int32, (T, TOPK), 1)
    idx_full = jnp.zeros((T, TOPK), jnp.int32)
    w_full = jnp.zeros((T, TOPK), jnp.float32)
    for k in range(TOPK):
        mk = jnp.max(sm, axis=1, keepdims=True)
        ik = jnp.min(jnp.where(sm == mk, col, _BIG), axis=1, keepdims=True)
        onehot = col == ik
        wk = jnp.sum(jnp.where(onehot, scores, 0.0), axis=1, keepdims=True)
        idx_full = idx_full + jnp.where(kcol == k, ik, 0)
        w_full = w_full + jnp.where(kcol == k, wk, 0.0)
        sm = jnp.where(onehot, _NEG, sm)

    w_full = w_full / jnp.sum(w_full, axis=1, keepdims=True) * RS
    idx_ref[...] = idx_full
    w_ref[...] = w_full


def _gate(x, gate_w, gate_b):
    return pl.pallas_call(
        _gate_body,
        out_shape=[jax.ShapeDtypeStruct((T, TOPK), jnp.int32),
                   jax.ShapeDtypeStruct((T, TOPK), jnp.float32)],
    )(x, gate_w, gate_b.reshape(1, E))


def _plan_body(idx_ref, pos_ref, posT_ref, te_ref):
    idx = idx_ref[...]
    col = lax.broadcasted_iota(jnp.int32, (T, E), 1)

    m_cnt = jnp.zeros((T, E), jnp.float32)
    for k in range(TOPK):
        m_cnt = m_cnt + (idx[:, k:k + 1] == col).astype(jnp.float32)

    r2 = lax.broadcasted_iota(jnp.int32, (T, T), 0)
    c2 = lax.broadcasted_iota(jnp.int32, (T, T), 1)
    tril = (r2 > c2).astype(jnp.float32)
    csum = lax.dot_general(tril, m_cnt, (((1,), (0,)), ((), ())),
                           preferred_element_type=jnp.float32,
                           precision=lax.Precision.HIGHEST)

    counts = (csum[T - 1:T, :] + m_cnt[T - 1:T, :]).astype(jnp.int32)
    padded = ((counts + (TILE - 1)) // TILE) * TILE

    re = lax.broadcasted_iota(jnp.int32, (E, E), 0)
    ce = lax.broadcasted_iota(jnp.int32, (E, E), 1)
    triu = (re < ce).astype(jnp.float32)
    off = lax.dot_general(padded.astype(jnp.float32), triu,
                          (((1,), (0,)), ((), ())),
                          preferred_element_type=jnp.float32,
                          precision=lax.Precision.HIGHEST)

    posmat = off + csum
    kcol = lax.broadcasted_iota(jnp.int32, (T, TOPK), 1)
    pos = jnp.zeros((T, TOPK), jnp.float32)
    for k in range(TOPK):
        sel = idx[:, k:k + 1] == col
        pk = jnp.sum(jnp.where(sel, posmat, 0.0), axis=1, keepdims=True)
        pos = pos + jnp.where(kcol == k, pk, 0.0)
    pos_ref[...] = pos.astype(jnp.int32)

    eye = (r2 == c2).astype(jnp.float32)
    posT = lax.dot_general(pos, eye, (((0,), (0,)), ((), ())),
                           preferred_element_type=jnp.float32,
                           precision=lax.Precision.HIGHEST)
    posT_ref[...] = posT.astype(jnp.int32)

    tcol = lax.broadcasted_iota(jnp.int32, (8, 256), 1) * TILE
    te = jnp.zeros((8, 256), jnp.int32)
    for e in range(E):
        te = te + (tcol.astype(jnp.float32) >= off[0, e]).astype(jnp.int32)
    te_ref[...] = te - 1


def _plan(indices):
    return pl.pallas_call(
        _plan_body,
        out_shape=[jax.ShapeDtypeStruct((T, TOPK), jnp.int32),
                   jax.ShapeDtypeStruct((TOPK, T), jnp.int32),
                   jax.ShapeDtypeStruct((8, 256), jnp.int32)],
    )(indices)


def _make_scatter():
    mesh = plsc.VectorSubcoreMesh(core_axis_name="c", subcore_axis_name="s")

    @functools.partial(
        pl.kernel,
        out_type=jax.ShapeDtypeStruct((P_PAD, D), jnp.float32),
        mesh=mesh,
        scratch_types=(
            [pltpu.VMEM((TPW, D), jnp.float32)]
            + [pltpu.VMEM((TPW,), jnp.int32) for _ in range(TOPK)]
            + [pltpu.SemaphoreType.DMA]
        ),
    )
    def scatter_x(x_hbm, posTf_hbm, xe_hbm, rows_v, *rest):
        idxs, sem = rest[:TOPK], rest[TOPK]
        wid = lax.axis_index("s") * 2 + lax.axis_index("c")
        base = wid * TPW
        pltpu.sync_copy(x_hbm.at[pl.ds(base, TPW)], rows_v)
        for k in range(TOPK):
            pltpu.sync_copy(posTf_hbm.at[pl.ds(k * T + base, TPW)], idxs[k])
        copies = []
        for k in range(TOPK):
            copies.append(
                pltpu.async_copy(rows_v, xe_hbm.at[idxs[k]], sem))
        for c in copies:
            c.wait()

    return scatter_x


@functools.cache
def _get_scatter():
    return _make_scatter()


def _scatter_x(x, pos_t):
    return _get_scatter()(x, pos_t)


def _mlp_body(te_ref, xe_ref, wgu_ref, wd_ref, ye_ref):
    x = xe_ref[...].astype(jnp.bfloat16)
    wgu = wgu_ref[0].astype(jnp.bfloat16)
    h = lax.dot_general(x, wgu, (((1,), (1,)), ((), ())),
                        preferred_element_type=jnp.float32)
    gate = h[:, :DFF]
    up = h[:, DFF:]
    act = (jax.nn.silu(gate) * up).astype(jnp.bfloat16)
    wd = wd_ref[0].astype(jnp.bfloat16)
    y = lax.dot_general(act, wd, (((1,), (1,)), ((), ())),
                        preferred_element_type=jnp.float32)
    ye_ref[...] = y


def _grouped_mlp(te_vec, xe, w_gate_up, w_down):
    grid_spec = pltpu.PrefetchScalarGridSpec(
        num_scalar_prefetch=1,
        grid=(NTILES,),
        in_specs=[
            pl.BlockSpec((TILE, D), lambda i, s: (i, 0)),
            pl.BlockSpec((1, 2 * DFF, D), lambda i, s: (s[i], 0, 0)),
            pl.BlockSpec((1, D, DFF), lambda i, s: (s[i], 0, 0)),
        ],
        out_specs=pl.BlockSpec((TILE, D), lambda i, s: (i, 0)),
    )
    return pl.pallas_call(
        _mlp_body,
        grid_spec=grid_spec,
        out_shape=jax.ShapeDtypeStruct((P_PAD, D), jnp.float32),
    )(te_vec, xe, w_gate_up, w_down)


def _make_gather():
    mesh = plsc.VectorSubcoreMesh(core_axis_name="c", subcore_axis_name="s")
    ppw = T * TOPK // NW
    chunk = 64

    @functools.partial(
        pl.kernel,
        out_type=jax.ShapeDtypeStruct((T * TOPK, D), jnp.float32),
        mesh=mesh,
        scratch_types=[
            pltpu.VMEM((chunk, D), jnp.float32),
            pltpu.VMEM((chunk,), jnp.int32),
            pltpu.SemaphoreType.DMA,
        ],
    )
    def gather_y(ye_hbm, posflat_hbm, yg_hbm, buf_v, idx_v, sem):
        wid = lax.axis_index("s") * 2 + lax.axis_index("c")
        for c in range(ppw // chunk):
            pbase = wid * ppw + c * chunk
            pltpu.sync_copy(posflat_hbm.at[pl.ds(pbase, chunk)], idx_v)
            pltpu.async_copy(ye_hbm.at[idx_v], buf_v, sem).wait()
            pltpu.sync_copy(buf_v, yg_hbm.at[pl.ds(pbase, chunk)])

    return gather_y


@functools.cache
def _get_gather():
    return _make_gather()


def _gather_y(ye, posflat):
    return _get_gather()(ye, posflat)


CTILE = 128


def _combine_body(x_ref, shgu_ref, shd_ref, yg_ref, w_ref, out_ref):
    x = x_ref[...].astype(jnp.bfloat16)
    acc = jnp.zeros((CTILE, D), jnp.float32)
    for s2 in range(NSH):
        wgu = shgu_ref[s2].astype(jnp.bfloat16)
        h = lax.dot_general(x, wgu, (((1,), (1,)), ((), ())),
                            preferred_element_type=jnp.float32)
        act = (jax.nn.silu(h[:, :DFF]) * h[:, DFF:]).astype(jnp.bfloat16)
        wd = shd_ref[s2].astype(jnp.bfloat16)
        acc = acc + lax.dot_general(act, wd, (((1,), (1,)), ((), ())),
                                    preferred_element_type=jnp.float32)
    w = w_ref[...]
    for k in range(TOPK):
        acc = acc + w[:, k:k + 1] * yg_ref[:, k, :]
    out_ref[...] = acc


def _combine(x, sh_gate_up, sh_down, yg, weights):
    return pl.pallas_call(
        _combine_body,
        grid=(T // CTILE,),
        in_specs=[
            pl.BlockSpec((CTILE, D), lambda i: (i, 0)),
            pl.BlockSpec((NSH, 2 * DFF, D), lambda i: (0, 0, 0)),
            pl.BlockSpec((NSH, D, DFF), lambda i: (0, 0, 0)),
            pl.BlockSpec((CTILE, TOPK, D), lambda i: (i, 0, 0)),
            pl.BlockSpec((CTILE, TOPK), lambda i: (i, 0)),
        ],
        out_specs=pl.BlockSpec((CTILE, D), lambda i: (i, 0)),
        out_shape=jax.ShapeDtypeStruct((T, D), jnp.float32),
    )(x, sh_gate_up, sh_down, yg, weights)


def kernel(hidden_states, gate_w, gate_b, w_gate_up, w_down, sh_gate_up,
           sh_down):
    orig_shape = hidden_states.shape
    x = hidden_states.reshape(-1, D)

    indices, weights = _gate(x, gate_w, gate_b)
    pos, pos_t, te = _plan(indices)
    te_vec = te.reshape(-1)[:NTILES]

    xe = _scatter_x(x, pos_t.reshape(-1))
    ye = _grouped_mlp(te_vec, xe, w_gate_up, w_down)
    yg = _gather_y(ye, pos.reshape(-1))

    out = _combine(x, sh_gate_up, sh_down,
                   yg.reshape(T, TOPK, D), weights)
    return out.reshape(orig_shape)


# speedup vs baseline: 2.2810x; 1.0256x over previous
import functools

import jax
import jax.numpy as jnp
from jax import lax
from jax.experimental import pallas as pl
from jax.experimental.pallas import tpu as pltpu
from jax.experimental.pallas import tpu_sc as plsc

D = 1024
DFF = 512
E = 64
NG = 8
GSZ = E // NG
TKG = 4
TOPK = 8
NSH = 2
RS = 2.5
T = 2048

TILE = 128
P_PAD = 16384 + E * TILE
NTILES = P_PAD // TILE
NW = 32
TPW = T // NW

_BIG = 1 << 30
_NEG = -1e30

DH = D // 2


def _pack_bf16(v):
    u = lax.bitcast_convert_type(v, jnp.uint32)
    r = u + jnp.uint32(0x7FFF) + ((u >> 16) & jnp.uint32(1))
    lo = (r[:, :DH] >> 16) & jnp.uint32(0xFFFF)
    hi = r[:, DH:] & jnp.uint32(0xFFFF0000)
    return lax.bitcast_convert_type(lo | hi, jnp.int32)


def _unpack_bf16(p):
    pu = lax.bitcast_convert_type(p, jnp.uint32)
    left = lax.bitcast_convert_type(pu << 16, jnp.float32)
    right = lax.bitcast_convert_type(pu & jnp.uint32(0xFFFF0000), jnp.float32)
    return left.astype(jnp.bfloat16), right.astype(jnp.bfloat16)


def _gate_body(x_ref, gw_ref, gb_ref, idx_ref, w_ref, xp_ref):
    x = x_ref[...]
    xp_ref[...] = _pack_bf16(x)
    z = lax.dot_general(x, gw_ref[...], (((1,), (1,)), ((), ())),
                        preferred_element_type=jnp.float32,
                        precision=lax.Precision.DEFAULT)
    scores = jax.nn.sigmoid(z)
    s = scores + gb_ref[...]

    col = lax.broadcasted_iota(jnp.int32, (T, E), 1)
    gid = col // GSZ

    gsc = jnp.zeros((T, NG), jnp.float32)
    gcol = lax.broadcasted_iota(jnp.int32, (T, NG), 1)
    for g in range(NG):
        m = jnp.where(gid == g, s, _NEG)
        m1 = jnp.max(m, axis=1, keepdims=True)
        i1 = jnp.min(jnp.where(m == m1, col, _BIG), axis=1, keepdims=True)
        m2 = jnp.max(jnp.where(col == i1, _NEG, m), axis=1, keepdims=True)
        gsc = gsc + jnp.where(gcol == g, m1 + m2, 0.0)

    sel_g = jnp.zeros((T, NG), jnp.bool_)
    gs = gsc
    for _ in range(TKG):
        mg = jnp.max(gs, axis=1, keepdims=True)
        ig = jnp.min(jnp.where(gs == mg, gcol, _BIG), axis=1, keepdims=True)
        sel_g = sel_g | (gcol == ig)
        gs = jnp.where(gcol == ig, _NEG, gs)

    allowed = jnp.zeros((T, E), jnp.bool_)
    for g in range(NG):
        allowed = allowed | ((gid == g) & sel_g[:, g:g + 1])

    sm = jnp.where(allowed, s, _NEG)
    kcol = lax.broadcasted_iota(jnp.int32, (T, TOPK), 1)
    idx_full = jnp.zeros((T, TOPK), jnp.int32)
    w_full = jnp.zeros((T, TOPK), jnp.float32)
    for k in range(TOPK):
        mk = jnp.max(sm, axis=1, keepdims=True)
        ik = jnp.min(jnp.where(sm == mk, col, _BIG), axis=1, keepdims=True)
        onehot = col == ik
        wk = jnp.sum(jnp.where(onehot, scores, 0.0), axis=1, keepdims=True)
        idx_full = idx_full + jnp.where(kcol == k, ik, 0)
        w_full = w_full + jnp.where(kcol == k, wk, 0.0)
        sm = jnp.where(onehot, _NEG, sm)

    w_full = w_full / jnp.sum(w_full, axis=1, keepdims=True) * RS
    idx_ref[...] = idx_full
    w_ref[...] = w_full


def _gate(x, gate_w, gate_b):
    return pl.pallas_call(
        _gate_body,
        out_shape=[jax.ShapeDtypeStruct((T, TOPK), jnp.int32),
                   jax.ShapeDtypeStruct((T, TOPK), jnp.float32),
                   jax.ShapeDtypeStruct((T, DH), jnp.int32)],
    )(x, gate_w, gate_b.reshape(1, E))


def _plan_body(idx_ref, pos_ref, posT_ref, te_ref):
    idx = idx_ref[...]
    col = lax.broadcasted_iota(jnp.int32, (T, E), 1)

    m_cnt = jnp.zeros((T, E), jnp.float32)
    for k in range(TOPK):
        m_cnt = m_cnt + (idx[:, k:k + 1] == col).astype(jnp.float32)

    r2 = lax.broadcasted_iota(jnp.int32, (T, T), 0)
    c2 = lax.broadcasted_iota(jnp.int32, (T, T), 1)
    tril = (r2 > c2).astype(jnp.float32)
    csum = lax.dot_general(tril, m_cnt, (((1,), (0,)), ((), ())),
                           preferred_element_type=jnp.float32,
                           precision=lax.Precision.HIGHEST)

    counts = (csum[T - 1:T, :] + m_cnt[T - 1:T, :]).astype(jnp.int32)
    padded = ((counts + (TILE - 1)) // TILE) * TILE

    re = lax.broadcasted_iota(jnp.int32, (E, E), 0)
    ce = lax.broadcasted_iota(jnp.int32, (E, E), 1)
    triu = (re < ce).astype(jnp.float32)
    off = lax.dot_general(padded.astype(jnp.float32), triu,
                          (((1,), (0,)), ((), ())),
                          preferred_element_type=jnp.float32,
                          precision=lax.Precision.HIGHEST)

    posmat = off + csum
    kcol = lax.broadcasted_iota(jnp.int32, (T, TOPK), 1)
    pos = jnp.zeros((T, TOPK), jnp.float32)
    for k in range(TOPK):
        sel = idx[:, k:k + 1] == col
        pk = jnp.sum(jnp.where(sel, posmat, 0.0), axis=1, keepdims=True)
        pos = pos + jnp.where(kcol == k, pk, 0.0)
    pos_ref[...] = pos.astype(jnp.int32)

    eye = (r2 == c2).astype(jnp.float32)
    posT = lax.dot_general(pos, eye, (((0,), (0,)), ((), ())),
                           preferred_element_type=jnp.float32,
                           precision=lax.Precision.HIGHEST)
    posT_ref[...] = posT.astype(jnp.int32)

    tcol = lax.broadcasted_iota(jnp.int32, (8, 256), 1) * TILE
    te = jnp.zeros((8, 256), jnp.int32)
    for e in range(E):
        te = te + (tcol.astype(jnp.float32) >= off[0, e]).astype(jnp.int32)
    te_ref[...] = te - 1


def _plan(indices):
    return pl.pallas_call(
        _plan_body,
        out_shape=[jax.ShapeDtypeStruct((T, TOPK), jnp.int32),
                   jax.ShapeDtypeStruct((TOPK, T), jnp.int32),
                   jax.ShapeDtypeStruct((8, 256), jnp.int32)],
    )(indices)


def _make_scatter():
    mesh = plsc.VectorSubcoreMesh(core_axis_name="c", subcore_axis_name="s")

    @functools.partial(
        pl.kernel,
        out_type=jax.ShapeDtypeStruct((P_PAD, DH), jnp.int32),
        mesh=mesh,
        scratch_types=(
            [pltpu.VMEM((TPW, DH), jnp.int32)]
            + [pltpu.VMEM((TPW,), jnp.int32) for _ in range(TOPK)]
            + [pltpu.SemaphoreType.DMA]
        ),
    )
    def scatter_x(x_hbm, posTf_hbm, xe_hbm, rows_v, *rest):
        idxs, sem = rest[:TOPK], rest[TOPK]
        wid = lax.axis_index("s") * 2 + lax.axis_index("c")
        base = wid * TPW
        pltpu.sync_copy(x_hbm.at[pl.ds(base, TPW)], rows_v)
        for k in range(TOPK):
            pltpu.sync_copy(posTf_hbm.at[pl.ds(k * T + base, TPW)], idxs[k])
        copies = []
        for k in range(TOPK):
            copies.append(
                pltpu.async_copy(rows_v, xe_hbm.at[idxs[k]], sem))
        for c in copies:
            c.wait()

    return scatter_x


@functools.cache
def _get_scatter():
    return _make_scatter()


def _scatter_x(x, pos_t):
    return _get_scatter()(x, pos_t)


def _mlp_body(te_ref, xe_ref, wgu_ref, wd_ref, ye_ref):
    xl, xr = _unpack_bf16(xe_ref[...])
    wgu = wgu_ref[0].astype(jnp.bfloat16)
    h = (lax.dot_general(xl, wgu[:, :DH], (((1,), (1,)), ((), ())),
                         preferred_element_type=jnp.float32)
         + lax.dot_general(xr, wgu[:, DH:], (((1,), (1,)), ((), ())),
                           preferred_element_type=jnp.float32))
    gate = h[:, :DFF]
    up = h[:, DFF:]
    act = (jax.nn.silu(gate) * up).astype(jnp.bfloat16)
    wd = wd_ref[0].astype(jnp.bfloat16)
    y = lax.dot_general(act, wd, (((1,), (1,)), ((), ())),
                        preferred_element_type=jnp.float32)
    ye_ref[...] = _pack_bf16(y)


def _grouped_mlp(te_vec, xe, w_gate_up, w_down):
    grid_spec = pltpu.PrefetchScalarGridSpec(
        num_scalar_prefetch=1,
        grid=(NTILES,),
        in_specs=[
            pl.BlockSpec((TILE, DH), lambda i, s: (i, 0)),
            pl.BlockSpec((1, 2 * DFF, D), lambda i, s: (s[i], 0, 0)),
            pl.BlockSpec((1, D, DFF), lambda i, s: (s[i], 0, 0)),
        ],
        out_specs=pl.BlockSpec((TILE, DH), lambda i, s: (i, 0)),
    )
    return pl.pallas_call(
        _mlp_body,
        grid_spec=grid_spec,
        out_shape=jax.ShapeDtypeStruct((P_PAD, DH), jnp.int32),
    )(te_vec, xe, w_gate_up, w_down)


def _make_gather():
    mesh = plsc.VectorSubcoreMesh(core_axis_name="c", subcore_axis_name="s")
    ppw = T * TOPK // NW
    chunk = 64

    @functools.partial(
        pl.kernel,
        out_type=jax.ShapeDtypeStruct((T * TOPK, DH), jnp.int32),
        mesh=mesh,
        scratch_types=[
            pltpu.VMEM((chunk, DH), jnp.int32),
            pltpu.VMEM((chunk,), jnp.int32),
            pltpu.SemaphoreType.DMA,
        ],
    )
    def gather_y(ye_hbm, posflat_hbm, yg_hbm, buf_v, idx_v, sem):
        wid = lax.axis_index("s") * 2 + lax.axis_index("c")
        for c in range(ppw // chunk):
            pbase = wid * ppw + c * chunk
            pltpu.sync_copy(posflat_hbm.at[pl.ds(pbase, chunk)], idx_v)
            pltpu.async_copy(ye_hbm.at[idx_v], buf_v, sem).wait()
            pltpu.sync_copy(buf_v, yg_hbm.at[pl.ds(pbase, chunk)])

    return gather_y


@functools.cache
def _get_gather():
    return _make_gather()


def _gather_y(ye, posflat):
    return _get_gather()(ye, posflat)


CTILE = 128


def _combine_body(x_ref, shgu_ref, shd_ref, yg_ref, w_ref, out_ref):
    x = x_ref[...].astype(jnp.bfloat16)
    acc = jnp.zeros((CTILE, D), jnp.float32)
    for s2 in range(NSH):
        wgu = shgu_ref[s2].astype(jnp.bfloat16)
        h = lax.dot_general(x, wgu, (((1,), (1,)), ((), ())),
                            preferred_element_type=jnp.float32)
        act = (jax.nn.silu(h[:, :DFF]) * h[:, DFF:]).astype(jnp.bfloat16)
        wd = shd_ref[s2].astype(jnp.bfloat16)
        acc = acc + lax.dot_general(act, wd, (((1,), (1,)), ((), ())),
                                    preferred_element_type=jnp.float32)
    w = w_ref[...]
    accl = acc[:, :DH]
    accr = acc[:, DH:]
    for k in range(TOPK):
        yl, yr = _unpack_bf16(yg_ref[:, k, :])
        wk = w[:, k:k + 1]
        accl = accl + wk * yl.astype(jnp.float32)
        accr = accr + wk * yr.astype(jnp.float32)
    out_ref[:, :DH] = accl
    out_ref[:, DH:] = accr


def _combine(x, sh_gate_up, sh_down, yg, weights):
    return pl.pallas_call(
        _combine_body,
        grid=(T // CTILE,),
        in_specs=[
            pl.BlockSpec((CTILE, D), lambda i: (i, 0)),
            pl.BlockSpec((NSH, 2 * DFF, D), lambda i: (0, 0, 0)),
            pl.BlockSpec((NSH, D, DFF), lambda i: (0, 0, 0)),
            pl.BlockSpec((CTILE, TOPK, DH), lambda i: (i, 0, 0)),
            pl.BlockSpec((CTILE, TOPK), lambda i: (i, 0)),
        ],
        out_specs=pl.BlockSpec((CTILE, D), lambda i: (i, 0)),
        out_shape=jax.ShapeDtypeStruct((T, D), jnp.float32),
    )(x, sh_gate_up, sh_down, yg, weights)


def kernel(hidden_states, gate_w, gate_b, w_gate_up, w_down, sh_gate_up,
           sh_down):
    orig_shape = hidden_states.shape
    x = hidden_states.reshape(-1, D)

    indices, weights, x_packed = _gate(x, gate_w, gate_b)
    pos, pos_t, te = _plan(indices)
    te_vec = te.reshape(-1)[:NTILES]

    xe = _scatter_x(x_packed, pos_t.reshape(-1))
    ye = _grouped_mlp(te_vec, xe, w_gate_up, w_down)
    yg = _gather_y(ye, pos.reshape(-1))

    out = _combine(x, sh_gate_up, sh_down,
                   yg.reshape(T, TOPK, DH), weights)
    return out.reshape(orig_shape)


# speedup vs baseline: 2.3402x; 1.0259x over previous
import functools

import jax
import jax.numpy as jnp
from jax import lax
from jax.experimental import pallas as pl
from jax.experimental.pallas import tpu as pltpu
from jax.experimental.pallas import tpu_sc as plsc

D = 1024
DFF = 512
E = 64
NG = 8
GSZ = E // NG
TKG = 4
TOPK = 8
NSH = 2
RS = 2.5
T = 2048

TILE = 128
P_PAD = 16384 + E * TILE
NTILES = P_PAD // TILE
NW = 32
TPW = T // NW

_BIG = 1 << 30
_NEG = -1e30

DH = D // 2


def _pack_bf16(v):
    u = lax.bitcast_convert_type(v, jnp.uint32)
    r = u + jnp.uint32(0x7FFF) + ((u >> 16) & jnp.uint32(1))
    lo = (r[:, :DH] >> 16) & jnp.uint32(0xFFFF)
    hi = r[:, DH:] & jnp.uint32(0xFFFF0000)
    return lax.bitcast_convert_type(lo | hi, jnp.int32)


def _unpack_bf16(p):
    pu = lax.bitcast_convert_type(p, jnp.uint32)
    left = lax.bitcast_convert_type(pu << 16, jnp.float32)
    right = lax.bitcast_convert_type(pu & jnp.uint32(0xFFFF0000), jnp.float32)
    return left.astype(jnp.bfloat16), right.astype(jnp.bfloat16)


def _gate_body(x_ref, gw_ref, gb_ref, w_ref, xp_ref, pos_ref, posT_ref,
               te_ref):
    x = x_ref[...]
    xp_ref[...] = _pack_bf16(x)
    z = lax.dot_general(x, gw_ref[...], (((1,), (1,)), ((), ())),
                        preferred_element_type=jnp.float32,
                        precision=lax.Precision.DEFAULT)
    scores = jax.nn.sigmoid(z)
    s = scores + gb_ref[...]

    col = lax.broadcasted_iota(jnp.int32, (T, E), 1)
    gid = col // GSZ

    gsc = jnp.zeros((T, NG), jnp.float32)
    gcol = lax.broadcasted_iota(jnp.int32, (T, NG), 1)
    for g in range(NG):
        m = jnp.where(gid == g, s, _NEG)
        m1 = jnp.max(m, axis=1, keepdims=True)
        i1 = jnp.min(jnp.where(m == m1, col, _BIG), axis=1, keepdims=True)
        m2 = jnp.max(jnp.where(col == i1, _NEG, m), axis=1, keepdims=True)
        gsc = gsc + jnp.where(gcol == g, m1 + m2, 0.0)

    sel_g = jnp.zeros((T, NG), jnp.bool_)
    gs = gsc
    for _ in range(TKG):
        mg = jnp.max(gs, axis=1, keepdims=True)
        ig = jnp.min(jnp.where(gs == mg, gcol, _BIG), axis=1, keepdims=True)
        sel_g = sel_g | (gcol == ig)
        gs = jnp.where(gcol == ig, _NEG, gs)

    allowed = jnp.zeros((T, E), jnp.bool_)
    for g in range(NG):
        allowed = allowed | ((gid == g) & sel_g[:, g:g + 1])

    sm = jnp.where(allowed, s, _NEG)
    kcol = lax.broadcasted_iota(jnp.int32, (T, TOPK), 1)
    idx_full = jnp.zeros((T, TOPK), jnp.int32)
    w_full = jnp.zeros((T, TOPK), jnp.float32)
    for k in range(TOPK):
        mk = jnp.max(sm, axis=1, keepdims=True)
        ik = jnp.min(jnp.where(sm == mk, col, _BIG), axis=1, keepdims=True)
        onehot = col == ik
        wk = jnp.sum(jnp.where(onehot, scores, 0.0), axis=1, keepdims=True)
        idx_full = idx_full + jnp.where(kcol == k, ik, 0)
        w_full = w_full + jnp.where(kcol == k, wk, 0.0)
        sm = jnp.where(onehot, _NEG, sm)

    w_full = w_full / jnp.sum(w_full, axis=1, keepdims=True) * RS
    w_ref[...] = w_full

    idx = idx_full

    m_cnt = jnp.zeros((T, E), jnp.float32)
    for k in range(TOPK):
        m_cnt = m_cnt + (idx[:, k:k + 1] == col).astype(jnp.float32)

    riota = lax.broadcasted_iota(jnp.int32, (T, 1), 0)
    ciota = lax.broadcasted_iota(jnp.int32, (1, T), 1)
    tril = (riota > ciota).astype(jnp.bfloat16)
    csum = lax.dot_general(tril, m_cnt.astype(jnp.bfloat16),
                           (((1,), (0,)), ((), ())),
                           preferred_element_type=jnp.float32)

    counts = (csum[T - 1:T, :] + m_cnt[T - 1:T, :]).astype(jnp.int32)
    padded = ((counts + (TILE - 1)) // TILE) * TILE

    re = lax.broadcasted_iota(jnp.int32, (E, E), 0)
    ce = lax.broadcasted_iota(jnp.int32, (E, E), 1)
    triu = (re < ce).astype(jnp.float32)
    off = lax.dot_general(padded.astype(jnp.float32), triu,
                          (((1,), (0,)), ((), ())),
                          preferred_element_type=jnp.float32,
                          precision=lax.Precision.HIGHEST)

    posmat = off + csum
    kcol = lax.broadcasted_iota(jnp.int32, (T, TOPK), 1)
    pos = jnp.zeros((T, TOPK), jnp.float32)
    for k in range(TOPK):
        sel = idx[:, k:k + 1] == col
        pk = jnp.sum(jnp.where(sel, posmat, 0.0), axis=1, keepdims=True)
        pos = pos + jnp.where(kcol == k, pk, 0.0)
    pos_ref[...] = pos.astype(jnp.int32)

    eye = (riota == ciota).astype(jnp.float32)
    posT = lax.dot_general(pos, eye, (((0,), (0,)), ((), ())),
                           preferred_element_type=jnp.float32,
                           precision=lax.Precision.HIGHEST)
    posT_ref[...] = posT.astype(jnp.int32)

    tcol = lax.broadcasted_iota(jnp.int32, (8, 256), 1) * TILE
    te = jnp.zeros((8, 256), jnp.int32)
    for e in range(E):
        te = te + (tcol.astype(jnp.float32) >= off[0, e]).astype(jnp.int32)
    te_ref[...] = te - 1


def _gate(x, gate_w, gate_b):
    return pl.pallas_call(
        _gate_body,
        out_shape=[jax.ShapeDtypeStruct((T, TOPK), jnp.float32),
                   jax.ShapeDtypeStruct((T, DH), jnp.int32),
                   jax.ShapeDtypeStruct((T, TOPK), jnp.int32),
                   jax.ShapeDtypeStruct((TOPK, T), jnp.int32),
                   jax.ShapeDtypeStruct((8, 256), jnp.int32)],
    )(x, gate_w, gate_b.reshape(1, E))


def _make_scatter():
    mesh = plsc.VectorSubcoreMesh(core_axis_name="c", subcore_axis_name="s")

    @functools.partial(
        pl.kernel,
        out_type=jax.ShapeDtypeStruct((P_PAD, DH), jnp.int32),
        mesh=mesh,
        scratch_types=(
            [pltpu.VMEM((TPW, DH), jnp.int32)]
            + [pltpu.VMEM((TPW,), jnp.int32) for _ in range(TOPK)]
            + [pltpu.SemaphoreType.DMA]
        ),
    )
    def scatter_x(x_hbm, posTf_hbm, xe_hbm, rows_v, *rest):
        idxs, sem = rest[:TOPK], rest[TOPK]
        wid = lax.axis_index("s") * 2 + lax.axis_index("c")
        base = wid * TPW
        pltpu.sync_copy(x_hbm.at[pl.ds(base, TPW)], rows_v)
        for k in range(TOPK):
            pltpu.sync_copy(posTf_hbm.at[pl.ds(k * T + base, TPW)], idxs[k])
        copies = []
        for k in range(TOPK):
            copies.append(
                pltpu.async_copy(rows_v, xe_hbm.at[idxs[k]], sem))
        for c in copies:
            c.wait()

    return scatter_x


@functools.cache
def _get_scatter():
    return _make_scatter()


def _scatter_x(x, pos_t):
    return _get_scatter()(x, pos_t)


def _mlp_body(te_ref, xe_ref, wgu_ref, wd_ref, ye_ref):
    xl, xr = _unpack_bf16(xe_ref[...])
    wgu = wgu_ref[0].astype(jnp.bfloat16)
    h = (lax.dot_general(xl, wgu[:, :DH], (((1,), (1,)), ((), ())),
                         preferred_element_type=jnp.float32)
         + lax.dot_general(xr, wgu[:, DH:], (((1,), (1,)), ((), ())),
                           preferred_element_type=jnp.float32))
    gate = h[:, :DFF]
    up = h[:, DFF:]
    act = (jax.nn.silu(gate) * up).astype(jnp.bfloat16)
    wd = wd_ref[0].astype(jnp.bfloat16)
    y = lax.dot_general(act, wd, (((1,), (1,)), ((), ())),
                        preferred_element_type=jnp.float32)
    ye_ref[...] = _pack_bf16(y)


def _grouped_mlp(te_vec, xe, w_gate_up, w_down):
    grid_spec = pltpu.PrefetchScalarGridSpec(
        num_scalar_prefetch=1,
        grid=(NTILES,),
        in_specs=[
            pl.BlockSpec((TILE, DH), lambda i, s: (i, 0)),
            pl.BlockSpec((1, 2 * DFF, D), lambda i, s: (s[i], 0, 0)),
            pl.BlockSpec((1, D, DFF), lambda i, s: (s[i], 0, 0)),
        ],
        out_specs=pl.BlockSpec((TILE, DH), lambda i, s: (i, 0)),
    )
    return pl.pallas_call(
        _mlp_body,
        grid_spec=grid_spec,
        out_shape=jax.ShapeDtypeStruct((P_PAD, DH), jnp.int32),
    )(te_vec, xe, w_gate_up, w_down)


def _make_gather():
    mesh = plsc.VectorSubcoreMesh(core_axis_name="c", subcore_axis_name="s")
    ppw = T * TOPK // NW
    chunk = 64

    @functools.partial(
        pl.kernel,
        out_type=jax.ShapeDtypeStruct((T * TOPK, DH), jnp.int32),
        mesh=mesh,
        scratch_types=[
            pltpu.VMEM((chunk, DH), jnp.int32),
            pltpu.VMEM((chunk,), jnp.int32),
            pltpu.SemaphoreType.DMA,
        ],
    )
    def gather_y(ye_hbm, posflat_hbm, yg_hbm, buf_v, idx_v, sem):
        wid = lax.axis_index("s") * 2 + lax.axis_index("c")
        for c in range(ppw // chunk):
            pbase = wid * ppw + c * chunk
            pltpu.sync_copy(posflat_hbm.at[pl.ds(pbase, chunk)], idx_v)
            pltpu.async_copy(ye_hbm.at[idx_v], buf_v, sem).wait()
            pltpu.sync_copy(buf_v, yg_hbm.at[pl.ds(pbase, chunk)])

    return gather_y


@functools.cache
def _get_gather():
    return _make_gather()


def _gather_y(ye, posflat):
    return _get_gather()(ye, posflat)


CTILE = 128


def _combine_body(x_ref, shgu_ref, shd_ref, yg_ref, w_ref, out_ref):
    x = x_ref[...].astype(jnp.bfloat16)
    acc = jnp.zeros((CTILE, D), jnp.float32)
    for s2 in range(NSH):
        wgu = shgu_ref[s2].astype(jnp.bfloat16)
        h = lax.dot_general(x, wgu, (((1,), (1,)), ((), ())),
                            preferred_element_type=jnp.float32)
        act = (jax.nn.silu(h[:, :DFF]) * h[:, DFF:]).astype(jnp.bfloat16)
        wd = shd_ref[s2].astype(jnp.bfloat16)
        acc = acc + lax.dot_general(act, wd, (((1,), (1,)), ((), ())),
                                    preferred_element_type=jnp.float32)
    w = w_ref[...]
    accl = acc[:, :DH]
    accr = acc[:, DH:]
    for k in range(TOPK):
        yl, yr = _unpack_bf16(yg_ref[:, k, :])
        wk = w[:, k:k + 1]
        accl = accl + wk * yl.astype(jnp.float32)
        accr = accr + wk * yr.astype(jnp.float32)
    out_ref[:, :DH] = accl
    out_ref[:, DH:] = accr


def _combine(x, sh_gate_up, sh_down, yg, weights):
    return pl.pallas_call(
        _combine_body,
        grid=(T // CTILE,),
        in_specs=[
            pl.BlockSpec((CTILE, D), lambda i: (i, 0)),
            pl.BlockSpec((NSH, 2 * DFF, D), lambda i: (0, 0, 0)),
            pl.BlockSpec((NSH, D, DFF), lambda i: (0, 0, 0)),
            pl.BlockSpec((CTILE, TOPK, DH), lambda i: (i, 0, 0)),
            pl.BlockSpec((CTILE, TOPK), lambda i: (i, 0)),
        ],
        out_specs=pl.BlockSpec((CTILE, D), lambda i: (i, 0)),
        out_shape=jax.ShapeDtypeStruct((T, D), jnp.float32),
    )(x, sh_gate_up, sh_down, yg, weights)


def kernel(hidden_states, gate_w, gate_b, w_gate_up, w_down, sh_gate_up,
           sh_down):
    orig_shape = hidden_states.shape
    x = hidden_states.reshape(-1, D)

    weights, x_packed, pos, pos_t, te = _gate(x, gate_w, gate_b)
    te_vec = te.reshape(-1)[:NTILES]

    xe = _scatter_x(x_packed, pos_t.reshape(-1))
    ye = _grouped_mlp(te_vec, xe, w_gate_up, w_down)
    yg = _gather_y(ye, pos.reshape(-1))

    out = _combine(x, sh_gate_up, sh_down,
                   yg.reshape(T, TOPK, DH), weights)
    return out.reshape(orig_shape)
